# baseline (speedup 1.0000x reference)
import functools

import jax
import jax.numpy as jnp
from jax import lax
from jax.experimental import pallas as pl
from jax.experimental.pallas import tpu as pltpu
from jax.experimental.pallas import tpu_sc as plsc


@functools.lru_cache(maxsize=None)
def _make_gather(V, D, batch, hist):
    info = plsc.get_sparse_core_info()
    NC, NS = info.num_cores, info.num_subcores
    NW = NC * NS
    assert batch % NW == 0
    rows_per_w = batch // NW

    mesh = plsc.VectorSubcoreMesh(core_axis_name="c", subcore_axis_name="s")

    assert hist % 2 == 0 and rows_per_w % 16 == 0 and D % 16 == 0

    @functools.partial(
        pl.kernel,
        mesh=mesh,
        out_type=jax.ShapeDtypeStruct((hist, D, batch), jnp.float32),
        compiler_params=pltpu.CompilerParams(
            use_tc_tiling_on_sc=False, needs_layout_passes=False),
        scratch_types=[
            pltpu.VMEM((rows_per_w, hist), jnp.int32),
            pltpu.VMEM((hist, rows_per_w), jnp.int32),
            pltpu.VMEM((2, rows_per_w, D), jnp.float32),
            pltpu.VMEM((D, rows_per_w), jnp.float32),
            pltpu.SemaphoreType.DMA,
            pltpu.SemaphoreType.DMA,
        ],
    )
    def k(idx_hbm, table_hbm, out_hbm, raw_v, idx_v, rows_v, stage_v,
          sem0, sem1):
        cid = lax.axis_index("c")
        sid = lax.axis_index("s")
        wid = sid * NC + cid
        base = wid * rows_per_w
        pltpu.sync_copy(idx_hbm.at[wid], raw_v)
        lanes = lax.iota(jnp.int32, 16)

        def idx_body(h, carry):
            col = jnp.full((16,), 0, jnp.int32) + h
            for kk in range(rows_per_w // 16):
                vals = plsc.load_gather(raw_v, [lanes + kk * 16, col])
                idx_v[h, pl.ds(kk * 16, 16)] = vals
            return carry

        lax.fori_loop(0, hist, idx_body, 0)

        def start(h, buf, sem):
            return pltpu.async_copy(
                table_hbm.at[idx_v.at[h]], rows_v.at[buf], sem)

        def wait(h, buf, sem):
            pltpu.make_async_copy(
                table_hbm.at[idx_v.at[h]], rows_v.at[buf], sem).wait()

        def emit(h, buf):
            src = rows_v.at[buf]

            def d_body(d, carry):
                col = jnp.full((16,), 0, jnp.int32) + d
                for kk in range(rows_per_w // 16):
                    vals = plsc.load_gather(src, [lanes + kk * 16, col])
                    stage_v[d, pl.ds(kk * 16, 16)] = vals
                return carry

            lax.fori_loop(0, D, d_body, 0)
            pltpu.sync_copy(
                stage_v, out_hbm.at[h].at[:, pl.ds(base, rows_per_w)])

        start(0, 0, sem0)

        def h_body(hh, carry):
            h0 = hh * 2
            start(h0 + 1, 1, sem1)
            wait(h0, 0, sem0)
            emit(h0, 0)

            @pl.when(hh + 1 < hist // 2)
            def _():
                start(h0 + 2, 0, sem0)

            wait(h0 + 1, 1, sem1)
            emit(h0 + 1, 1)
            return carry

        lax.fori_loop(0, hist // 2, h_body, 0)

    return k, NW, rows_per_w


def kernel(x, table):
    batch, hist = x.shape
    V, D = table.shape
    k, NW, rows_per_w = _make_gather(V, D, batch, hist)
    idx = x.astype(jnp.int32).reshape(NW, rows_per_w, hist)
    return k(idx, table).transpose(2, 0, 1)


# speedup vs baseline: 1.3224x; 1.3224x over previous
import functools

import jax
import jax.numpy as jnp
from jax import lax
from jax.experimental import pallas as pl
from jax.experimental.pallas import tpu as pltpu
from jax.experimental.pallas import tpu_sc as plsc


@functools.lru_cache(maxsize=None)
def _make_gather(V, D, batch, hist):
    info = plsc.get_sparse_core_info()
    NC, NS = info.num_cores, info.num_subcores
    NW = NC * NS
    assert batch % NW == 0
    rows_per_w = batch // NW

    mesh = plsc.VectorSubcoreMesh(core_axis_name="c", subcore_axis_name="s")

    assert hist % 2 == 0 and rows_per_w % 16 == 0 and D % 16 == 0

    @functools.partial(
        pl.kernel,
        mesh=mesh,
        out_type=jax.ShapeDtypeStruct((hist, batch, D), jnp.float32),
        compiler_params=pltpu.CompilerParams(
            use_tc_tiling_on_sc=False, needs_layout_passes=False),
        scratch_types=[
            pltpu.VMEM((rows_per_w, hist), jnp.int32),
            pltpu.VMEM((hist, rows_per_w), jnp.int32),
            pltpu.VMEM((2, rows_per_w, D), jnp.float32),
            pltpu.VMEM((D, rows_per_w), jnp.float32),
            pltpu.SemaphoreType.DMA,
            pltpu.SemaphoreType.DMA,
        ],
    )
    def k(idx_hbm, table_hbm, out_hbm, raw_v, idx_v, rows_v, stage_v,
          sem0, sem1):
        cid = lax.axis_index("c")
        sid = lax.axis_index("s")
        wid = sid * NC + cid
        base = wid * rows_per_w
        pltpu.sync_copy(idx_hbm.at[wid], raw_v)
        lanes = lax.iota(jnp.int32, 16)

        def idx_body(h, carry):
            col = jnp.full((16,), 0, jnp.int32) + h
            for kk in range(rows_per_w // 16):
                vals = plsc.load_gather(raw_v, [lanes + kk * 16, col])
                idx_v[h, pl.ds(kk * 16, 16)] = vals
            return carry

        lax.fori_loop(0, hist, idx_body, 0)

        def start(h, buf, sem):
            return pltpu.async_copy(
                table_hbm.at[idx_v.at[h]], rows_v.at[buf], sem)

        def wait(h, buf, sem):
            pltpu.make_async_copy(
                table_hbm.at[idx_v.at[h]], rows_v.at[buf], sem).wait()

        def emit(h, buf):
            pltpu.sync_copy(
                rows_v.at[buf], out_hbm.at[h].at[pl.ds(base, rows_per_w)])

        start(0, 0, sem0)

        def h_body(hh, carry):
            h0 = hh * 2
            start(h0 + 1, 1, sem1)
            wait(h0, 0, sem0)
            emit(h0, 0)

            @pl.when(hh + 1 < hist // 2)
            def _():
                start(h0 + 2, 0, sem0)

            wait(h0 + 1, 1, sem1)
            emit(h0 + 1, 1)
            return carry

        lax.fori_loop(0, hist // 2, h_body, 0)

    return k, NW, rows_per_w


def kernel(x, table):
    batch, hist = x.shape
    V, D = table.shape
    k, NW, rows_per_w = _make_gather(V, D, batch, hist)
    idx = x.astype(jnp.int32).reshape(NW, rows_per_w, hist)
    return k(idx, table).transpose(1, 0, 2)
